# baseline (speedup 1.0000x reference)
import functools
import math

import jax
import jax.numpy as jnp
from jax import lax
from jax.experimental import pallas as pl
from jax.experimental.pallas import tpu as pltpu
from jax.experimental.pallas import tpu_sc as plsc

_D = 64
_LANES = 16
_NC = 2
_NS = 16
_NW = _NC * _NS
_GRP = 128
_K = 4
_SCALE = math.sqrt(_D)


def _body(x_ref, table_ref, out_ref, idx_v, gbuf, sbuf, gsems, osems):
    wid = lax.axis_index("s") * _NC + lax.axis_index("c")
    n_groups = x_ref.shape[0] // _NW
    gbase = wid * n_groups

    pltpu.sync_copy(x_ref.at[pl.ds(gbase, n_groups)], idx_v)

    def start_gather(b, g):
        pltpu.async_copy(table_ref.at[idx_v.at[g]], gbuf.at[b], gsems.at[b])

    def wait_gather(b, g):
        pltpu.make_async_copy(
            table_ref.at[idx_v.at[g]], gbuf.at[b], gsems.at[b]
        ).wait()

    def start_out(b, g):
        pltpu.async_copy(
            sbuf.at[b], out_ref.at[pl.ds((gbase + g) * _GRP, _GRP)], osems.at[b]
        )

    def wait_out(b):
        pltpu.make_async_copy(
            sbuf.at[b], out_ref.at[pl.ds(0, _GRP)], osems.at[b]
        ).wait()

    for b in range(_K):
        start_gather(b, b)

    @pl.loop(0, n_groups // _K)
    def _rounds(r):
        for b in range(_K):
            g = r * _K + b
            wait_gather(b, g)

            @pl.when(g >= _K)
            def _():
                wait_out(b)

            @pl.loop(0, _GRP, unroll=4)
            def _scale(row):
                for v in range(_D // _LANES):
                    sl = pl.ds(v * _LANES, _LANES)
                    sbuf[b, row, sl] = gbuf[b, row, sl] * _SCALE

            @pl.when(g + _K < x_ref.shape[0] // _NW)
            def _():
                start_gather(b, g + _K)

            start_out(b, g)

    for b in range(_K):
        wait_out(b)


@functools.partial(jax.jit, static_argnames=())
def kernel(x, table):
    batch, seq = x.shape
    n = batch * seq
    x2 = x.astype(jnp.int32).reshape(n // _GRP, _GRP)
    n_groups = n // _GRP // _NW

    fn = pl.kernel(
        _body,
        out_type=jax.ShapeDtypeStruct((n, _D), jnp.float32),
        mesh=plsc.VectorSubcoreMesh(core_axis_name="c", subcore_axis_name="s"),
        scratch_types=[
            pltpu.VMEM((n_groups, _GRP), jnp.int32),
            pltpu.VMEM((_K, _GRP, _D), jnp.float32),
            pltpu.VMEM((_K, _GRP, _D), jnp.float32),
            pltpu.SemaphoreType.DMA((_K,)),
            pltpu.SemaphoreType.DMA((_K,)),
        ],
        compiler_params=pltpu.CompilerParams(use_tc_tiling_on_sc=False),
    )
    out = fn(x2, table)
    return out.reshape(batch, seq, _D)
